# speedup vs baseline: 1.0467x; 1.0467x over previous
import jax
import jax.numpy as jnp
from jax.experimental import pallas as pl
from jax.experimental.pallas import tpu as pltpu


def _body(x_ref, ph_ref, pw_ref, o_ref, pe_ref):
    @pl.when(pl.program_id(0) == 0)
    def _():
        pe_ref[...] = ph_ref[...][:, None, :] + pw_ref[...][None, :, :]

    o_ref[...] = x_ref[...] + pe_ref[...][None, :, :, :]


def kernel(x, pos_height, pos_width):
    B, H, W, D = x.shape
    return pl.pallas_call(
        _body,
        grid=(B,),
        in_specs=[
            pl.BlockSpec((1, H, W, D), lambda b: (b, 0, 0, 0)),
            pl.BlockSpec((H, D), lambda b: (0, 0)),
            pl.BlockSpec((W, D), lambda b: (0, 0)),
        ],
        out_specs=pl.BlockSpec((1, H, W, D), lambda b: (b, 0, 0, 0)),
        out_shape=jax.ShapeDtypeStruct((B, H, W, D), x.dtype),
        scratch_shapes=[pltpu.VMEM((H, W, D), jnp.float32)],
    )(x, pos_height, pos_width)
